# baseline (speedup 1.0000x reference)
import functools

import jax
import jax.numpy as jnp
from jax import lax
from jax.experimental import pallas as pl
from jax.experimental.pallas import tpu as pltpu
from jax.experimental.pallas import tpu_sc as plsc

_IDX_CHUNK = 128


@functools.cache
def _make_gather(B, F):
    info = plsc.get_sparse_core_info()
    nc, ns = info.num_cores, info.num_subcores
    nw = nc * ns
    b_per_w = B // nw
    n_chunks = b_per_w // _IDX_CHUNK
    mesh = plsc.VectorSubcoreMesh(core_axis_name="c", subcore_axis_name="s")

    @functools.partial(
        pl.kernel,
        mesh=mesh,
        compiler_params=pltpu.CompilerParams(use_tc_tiling_on_sc=False),
        out_type=(
            jax.ShapeDtypeStruct((B, F), jnp.float32),
            jax.ShapeDtypeStruct((B, F), jnp.float32),
        ),
        scratch_types=[
            pltpu.VMEM((n_chunks, _IDX_CHUNK), jnp.int32),
            pltpu.VMEM((n_chunks, _IDX_CHUNK), jnp.int32),
            pltpu.VMEM((b_per_w, F), jnp.float32),
            pltpu.VMEM((b_per_w, F), jnp.float32),
            pltpu.SemaphoreType.DMA,
            pltpu.SemaphoreType.DMA,
        ],
    )
    def gather_k(user_hbm, item_hbm, uemb_hbm, iemb_hbm, u_out, i_out,
                 uidx_v, iidx_v, urows_v, irows_v, usem, isem):
        wid = lax.axis_index("s") * nc + lax.axis_index("c")
        base = wid * b_per_w
        for j in range(n_chunks):
            pltpu.sync_copy(user_hbm.at[pl.ds(base + j * _IDX_CHUNK,
                                              _IDX_CHUNK)], uidx_v.at[j])
            pltpu.sync_copy(item_hbm.at[pl.ds(base + j * _IDX_CHUNK,
                                              _IDX_CHUNK)], iidx_v.at[j])
        copies = []
        for j in range(n_chunks):
            copies.append(pltpu.async_copy(
                uemb_hbm.at[uidx_v.at[j]],
                urows_v.at[pl.ds(j * _IDX_CHUNK, _IDX_CHUNK)], usem))
            copies.append(pltpu.async_copy(
                iemb_hbm.at[iidx_v.at[j]],
                irows_v.at[pl.ds(j * _IDX_CHUNK, _IDX_CHUNK)], isem))
        for c in copies:
            c.wait()
        pltpu.sync_copy(urows_v, u_out.at[pl.ds(base, b_per_w)])
        pltpu.sync_copy(irows_v, i_out.at[pl.ds(base, b_per_w)])

    return gather_k


def _mlp_pallas(u_rows, i_rows, W1u, W1i, b1, W2, b2, W3, b3):
    B, F = u_rows.shape
    blk = 2048

    def mlp_body(u_ref, i_ref, w1u_ref, w1i_ref, b1_ref, w2_ref, b2_ref,
                 w3_ref, b3_ref, out_ref):
        h = (jnp.dot(u_ref[...], w1u_ref[...],
                     preferred_element_type=jnp.float32)
             + jnp.dot(i_ref[...], w1i_ref[...],
                       preferred_element_type=jnp.float32)
             + b1_ref[...])
        h = jnp.dot(h, w2_ref[...], preferred_element_type=jnp.float32) \
            + b2_ref[...]
        o = jnp.dot(h, w3_ref[...], preferred_element_type=jnp.float32) \
            + b3_ref[...]
        out_ref[...] = 1.0 / (1.0 + jnp.exp(-o))

    n1 = W1u.shape[1]
    n2 = W2.shape[1]
    return pl.pallas_call(
        mlp_body,
        grid=(B // blk,),
        in_specs=[
            pl.BlockSpec((blk, F), lambda i: (i, 0)),
            pl.BlockSpec((blk, F), lambda i: (i, 0)),
            pl.BlockSpec((F, n1), lambda i: (0, 0)),
            pl.BlockSpec((F, n1), lambda i: (0, 0)),
            pl.BlockSpec((1, n1), lambda i: (0, 0)),
            pl.BlockSpec((n1, n2), lambda i: (0, 0)),
            pl.BlockSpec((1, n2), lambda i: (0, 0)),
            pl.BlockSpec((n2, 1), lambda i: (0, 0)),
            pl.BlockSpec((1, 1), lambda i: (0, 0)),
        ],
        out_specs=pl.BlockSpec((blk, 1), lambda i: (i, 0)),
        out_shape=jax.ShapeDtypeStruct((B, 1), jnp.float32),
    )(u_rows, i_rows, W1u, W1i, b1.reshape(1, n1), W2, b2.reshape(1, n2),
      W3, b3.reshape(1, 1))


def kernel(user, item, user_emb, item_emb, W1, b1, W2, b2, W3, b3):
    B = user.shape[0]
    F = user_emb.shape[1]
    gather = _make_gather(B, F)
    u_rows, i_rows = gather(user.astype(jnp.int32), item.astype(jnp.int32),
                            user_emb, item_emb)
    return _mlp_pallas(u_rows, i_rows, W1[:F], W1[F:], b1, W2, b2, W3, b3)
